# baseline (speedup 1.0000x reference)
import functools

import jax
import jax.numpy as jnp
from jax import lax
from jax.experimental import pallas as pl
from jax.experimental.pallas import tpu as pltpu
from jax.experimental.pallas import tpu_sc as plsc

_NUM_CORES = 2
_NUM_SUBCORES = 16
_NW = _NUM_CORES * _NUM_SUBCORES
_CHUNK = 128
_CH = 192


def _wid():
    return lax.axis_index("s") * _NUM_CORES + lax.axis_index("c")


def _widen_rows(src, dst, n, D):
    def row(i):
        for r in range(2):
            for kk in range(D // 16):
                sl = pl.ds(kk * 16, 16)
                dst[2 * i + r, sl] = src[2 * i + r, sl]

    pl.loop(0, n // 2)(row)


def _make_pad(V, D):
    rows_w = (V - 1) // _NW // 8 * 8
    rem_off = rows_w * _NW
    rem_n = V - rem_off
    V_pad = (V + 7) // 8 * 8
    n_main = rows_w // _CH // 2 * 2
    tail_off = n_main * _CH
    tail_n = rows_w - tail_off
    assert 0 < tail_n <= _CH and tail_n % 2 == 0 and n_main % 2 == 0
    mesh = plsc.VectorSubcoreMesh(core_axis_name="c", subcore_axis_name="s")

    @functools.partial(
        pl.kernel,
        mesh=mesh,
        out_type=jax.ShapeDtypeStruct((V_pad, 2 * D), jnp.float32),
        scratch_types=[
            pltpu.VMEM((_CH, D), jnp.float32),
            pltpu.VMEM((_CH, D), jnp.float32),
            pltpu.VMEM((_CH, 2 * D), jnp.float32),
            pltpu.VMEM((_CH, 2 * D), jnp.float32),
            pltpu.SemaphoreType.DMA,
            pltpu.SemaphoreType.DMA,
        ],
    )
    def k(table_hbm, out2_hbm, b64_0, b64_1, b128_0, b128_1, sin, sout):
        wid = _wid()
        r0 = wid * rows_w
        b64 = (b64_0, b64_1)
        b128 = (b128_0, b128_1)

        def fire_in(c, b, n=_CH):
            off = pl.multiple_of(r0 + c * _CH, 8)
            pltpu.async_copy(
                table_hbm.at[pl.ds(off, n)], b64[b].at[pl.ds(0, n)], sin
            )

        def wait_in(b, n=_CH):
            pltpu.make_async_copy(
                table_hbm.at[pl.ds(0, n)], b64[b].at[pl.ds(0, n)], sin
            ).wait()

        def fire_out(c, b, n=_CH):
            off = pl.multiple_of(r0 + c * _CH, 8)
            pltpu.async_copy(
                b128[b].at[pl.ds(0, n)], out2_hbm.at[pl.ds(off, n)], sout
            )

        def wait_out(b, n=_CH):
            pltpu.make_async_copy(
                b128[b].at[pl.ds(0, n)], out2_hbm.at[pl.ds(0, n)], sout
            ).wait()

        fire_in(0, 0)

        def outer(g):
            for b in range(2):
                c = 2 * g + b
                fire_in(c + 1, (b + 1) % 2)
                wait_in(b)

                @pl.when(c >= 2)
                def _():
                    wait_out(b)

                _widen_rows(b64[b], b128[b], _CH, D)
                fire_out(c, b)

        pl.loop(0, n_main // 2)(outer)

        wait_in(0)
        wait_out(0)
        _widen_rows(b64[0], b128[0], tail_n, D)
        fire_out(n_main, 0, tail_n)
        wait_out(1)
        wait_out(0, tail_n)

        @pl.when(wid == 0)
        def _():
            rem_slab = rem_n // 8 * 8
            if rem_slab:
                pltpu.sync_copy(
                    table_hbm.at[pl.ds(rem_off, rem_slab)],
                    b64_1.at[pl.ds(0, rem_slab)],
                )
            for i in range(rem_n - rem_slab):
                pltpu.sync_copy(
                    table_hbm.at[pl.ds(rem_off + rem_slab + i, 1)],
                    b64_1.at[pl.ds(rem_slab + i, 1)],
                )
            _widen_rows(b64_1, b128_1, rem_n + (rem_n % 2), D)
            if rem_slab:
                pltpu.sync_copy(
                    b128_1.at[pl.ds(0, rem_slab)],
                    out2_hbm.at[pl.ds(rem_off, rem_slab)],
                )
            for i in range(rem_n - rem_slab):
                pltpu.sync_copy(
                    b128_1.at[pl.ds(rem_slab + i, 1)],
                    out2_hbm.at[pl.ds(rem_off + rem_slab + i, 1)],
                )

    return k


def _make_gather(B, V_pad, D):
    b_per_w = B // _NW
    n_chunks = b_per_w // _CHUNK
    half = b_per_w // 2
    mesh = plsc.VectorSubcoreMesh(core_axis_name="c", subcore_axis_name="s")

    @functools.partial(
        pl.kernel,
        mesh=mesh,
        out_type=jax.ShapeDtypeStruct((B, D), jnp.float32),
        scratch_types=[
            pltpu.VMEM((b_per_w,), jnp.int32),
            pltpu.VMEM((b_per_w, 2 * D), jnp.float32),
            pltpu.VMEM((half, D), jnp.float32),
            pltpu.SemaphoreType.DMA,
        ],
    )
    def k(labels_hbm, out2_hbm, out_hbm, lab_v, rows2, outbuf, sem):
        wid = _wid()
        base = wid * b_per_w
        pltpu.sync_copy(labels_hbm.at[pl.ds(base, b_per_w)], lab_v)
        copies = [
            pltpu.async_copy(
                out2_hbm.at[lab_v.at[pl.ds(j * _CHUNK, _CHUNK)]],
                rows2.at[pl.ds(j * _CHUNK, _CHUNK)],
                sem,
            )
            for j in range(n_chunks)
        ]
        for c in copies:
            c.wait()
        for h in range(2):
            def row(i, h=h):
                for kk in range(D // 16):
                    outbuf[i, pl.ds(kk * 16, 16)] = rows2[
                        h * half + i, pl.ds(kk * 16, 16)
                    ]

            pl.loop(0, half)(row)
            pltpu.sync_copy(
                outbuf, out_hbm.at[pl.ds(base + h * half, half)]
            )

    return k


def kernel(labels, embedding_table):
    B, = labels.shape
    V, D = embedding_table.shape
    padded = _make_pad(V, D)(embedding_table)
    return _make_gather(B, padded.shape[0], D)(
        labels.astype(jnp.int32), padded
    )


# speedup vs baseline: 1.9692x; 1.9692x over previous
import functools

import jax
import jax.numpy as jnp
from jax import lax
from jax.experimental import pallas as pl
from jax.experimental.pallas import tpu as pltpu
from jax.experimental.pallas import tpu_sc as plsc

_NUM_CORES = 2
_NUM_SUBCORES = 16
_NW = _NUM_CORES * _NUM_SUBCORES
_UNROLL = 16


def _make_gather(B, V, D):
    b_per_w = B // _NW
    mesh = plsc.VectorSubcoreMesh(core_axis_name="c", subcore_axis_name="s")

    @functools.partial(
        pl.kernel,
        mesh=mesh,
        out_type=jax.ShapeDtypeStruct((B, D), jnp.float32),
        scratch_types=[
            pltpu.VMEM((b_per_w,), jnp.int32),
            pltpu.VMEM((b_per_w, D), jnp.float32),
            pltpu.SemaphoreType.DMA,
        ],
    )
    def k(labels_hbm, table_hbm, out_hbm, idx_v, rows_v, sem):
        wid = lax.axis_index("s") * _NUM_CORES + lax.axis_index("c")
        base = wid * b_per_w
        pltpu.sync_copy(labels_hbm.at[pl.ds(base, b_per_w)], idx_v)

        def body(i):
            v = idx_v[pl.ds(i * _UNROLL, _UNROLL)]
            for j in range(_UNROLL):
                pltpu.async_copy(
                    table_hbm.at[pl.ds(v[j], 1)],
                    rows_v.at[pl.ds(i * _UNROLL + j, 1)],
                    sem,
                )

        pl.loop(0, b_per_w // _UNROLL)(body)
        pltpu.make_async_copy(
            table_hbm.at[pl.ds(0, b_per_w)], rows_v, sem
        ).wait()
        pltpu.sync_copy(rows_v, out_hbm.at[pl.ds(base, b_per_w)])

    return k


def kernel(labels, embedding_table):
    B, = labels.shape
    V, D = embedding_table.shape
    return _make_gather(B, V, D)(labels.astype(jnp.int32), embedding_table)
